# Initial kernel scaffold; baseline (speedup 1.0000x reference)
import jax
import jax.numpy as jnp
from jax.experimental import pallas as pl


def kernel(x, w_kn, b):
    raise NotImplementedError("write your pallas kernel here")


# speedup vs baseline: 2.4920x; 2.4920x over previous
import jax
import jax.numpy as jnp
from jax.experimental import pallas as pl
from jax.experimental.pallas import tpu as pltpu

_K = 1024
_N = 32000
_TM = 512
_TN = 3200


def _matmul_bias_kernel(x_ref, w_ref, b_ref, o_ref):
    acc = jnp.dot(x_ref[...], w_ref[...], preferred_element_type=jnp.float32)
    o_ref[...] = acc + b_ref[...]


def kernel(x, w_kn, b):
    lead_shape = x.shape[:-1]
    x2d = x.reshape(-1, _K).astype(jnp.bfloat16)
    M = x2d.shape[0]
    w_bf = w_kn[:, :_N].astype(jnp.bfloat16)
    b_sl = b[:, :_N]

    grid = (_N // _TN, M // _TM)

    out = pl.pallas_call(
        _matmul_bias_kernel,
        out_shape=jax.ShapeDtypeStruct((M, _N), jnp.float32),
        grid=grid,
        in_specs=[
            pl.BlockSpec((_TM, _K), lambda j, i: (i, 0)),
            pl.BlockSpec((_K, _TN), lambda j, i: (0, j)),
            pl.BlockSpec((1, _TN), lambda j, i: (0, j)),
        ],
        out_specs=pl.BlockSpec((_TM, _TN), lambda j, i: (i, j)),
        compiler_params=pltpu.CompilerParams(
            dimension_semantics=("parallel", "parallel"),
            vmem_limit_bytes=64 * 1024 * 1024,
        ),
        cost_estimate=pl.CostEstimate(
            flops=2 * M * _N * _K,
            transcendentals=0,
            bytes_accessed=(
                x2d.size * 2 + w_bf.size * 2 + b_sl.size * 4 + M * _N * 4
            ),
        ),
    )(x2d, w_bf, b_sl)

    return out.reshape(*lead_shape, _N)


# speedup vs baseline: 2.6626x; 1.0685x over previous
import jax
import jax.numpy as jnp
from jax.experimental import pallas as pl
from jax.experimental.pallas import tpu as pltpu

_K = 1024
_N = 32000
_TM = 1024
_TN = 3200


def _matmul_bias_kernel(x_ref, w_ref, b_ref, o_ref):
    acc = jnp.dot(x_ref[...], w_ref[...], preferred_element_type=jnp.float32)
    o_ref[...] = acc + b_ref[...]


def kernel(x, w_kn, b):
    lead_shape = x.shape[:-1]
    x2d = x.reshape(-1, _K).astype(jnp.bfloat16)
    M = x2d.shape[0]
    w_bf = w_kn[:, :_N].astype(jnp.bfloat16)
    b_sl = b[:, :_N]

    grid = (_N // _TN, M // _TM)

    out = pl.pallas_call(
        _matmul_bias_kernel,
        out_shape=jax.ShapeDtypeStruct((M, _N), jnp.float32),
        grid=grid,
        in_specs=[
            pl.BlockSpec((_TM, _K), lambda j, i: (i, 0)),
            pl.BlockSpec((_K, _TN), lambda j, i: (0, j)),
            pl.BlockSpec((1, _TN), lambda j, i: (0, j)),
        ],
        out_specs=pl.BlockSpec((_TM, _TN), lambda j, i: (i, j)),
        compiler_params=pltpu.CompilerParams(
            dimension_semantics=("parallel", "parallel"),
            vmem_limit_bytes=64 * 1024 * 1024,
        ),
        cost_estimate=pl.CostEstimate(
            flops=2 * M * _N * _K,
            transcendentals=0,
            bytes_accessed=(
                x2d.size * 2 + w_bf.size * 2 + b_sl.size * 4 + M * _N * 4
            ),
        ),
    )(x2d, w_bf, b_sl)

    return out.reshape(*lead_shape, _N)


# speedup vs baseline: 2.7576x; 1.0357x over previous
import jax
import jax.numpy as jnp
from jax.experimental import pallas as pl
from jax.experimental.pallas import tpu as pltpu

_K = 1024
_N = 32000
_TM = 1024
_TN = 3200


def _matmul_bias_kernel(x_ref, w_ref, b_ref, o_ref):
    w_bf = w_ref[...].astype(jnp.bfloat16)
    acc = jnp.dot(x_ref[...], w_bf, preferred_element_type=jnp.float32)
    o_ref[...] = acc + b_ref[...]


def kernel(x, w_kn, b):
    lead_shape = x.shape[:-1]
    x2d = x.reshape(-1, _K).astype(jnp.bfloat16)
    M = x2d.shape[0]
    w_bf = w_kn
    b_sl = b[:, :_N]

    grid = (_N // _TN, M // _TM)

    out = pl.pallas_call(
        _matmul_bias_kernel,
        out_shape=jax.ShapeDtypeStruct((M, _N), jnp.float32),
        grid=grid,
        in_specs=[
            pl.BlockSpec((_TM, _K), lambda j, i: (i, 0)),
            pl.BlockSpec((_K, _TN), lambda j, i: (0, j)),
            pl.BlockSpec((1, _TN), lambda j, i: (0, j)),
        ],
        out_specs=pl.BlockSpec((_TM, _TN), lambda j, i: (i, j)),
        compiler_params=pltpu.CompilerParams(
            dimension_semantics=("parallel", "parallel"),
            vmem_limit_bytes=64 * 1024 * 1024,
        ),
        cost_estimate=pl.CostEstimate(
            flops=2 * M * _N * _K,
            transcendentals=0,
            bytes_accessed=(
                x2d.size * 2 + _K * _N * 4 + b_sl.size * 4 + M * _N * 4
            ),
        ),
    )(x2d, w_bf, b_sl)

    return out.reshape(*lead_shape, _N)


# speedup vs baseline: 2.7750x; 1.0063x over previous
import jax
import jax.numpy as jnp
from jax.experimental import pallas as pl
from jax.experimental.pallas import tpu as pltpu

_K = 1024
_N = 32000
_TM = 1024
_TN = 3200


def _matmul_bias_kernel(x_ref, w_ref, b_ref, o_ref):
    x_bf = x_ref[...].astype(jnp.bfloat16)
    w_bf = w_ref[...].astype(jnp.bfloat16)
    acc = jnp.dot(x_bf, w_bf, preferred_element_type=jnp.float32)
    o_ref[...] = acc + b_ref[...]


def kernel(x, w_kn, b):
    lead_shape = x.shape[:-1]
    x2d = x.reshape(-1, _K)
    M = x2d.shape[0]
    w_bf = w_kn
    b_sl = b[:, :_N]

    grid = (_N // _TN, M // _TM)

    out = pl.pallas_call(
        _matmul_bias_kernel,
        out_shape=jax.ShapeDtypeStruct((M, _N), jnp.float32),
        grid=grid,
        in_specs=[
            pl.BlockSpec((_TM, _K), lambda j, i: (i, 0)),
            pl.BlockSpec((_K, _TN), lambda j, i: (0, j)),
            pl.BlockSpec((1, _TN), lambda j, i: (0, j)),
        ],
        out_specs=pl.BlockSpec((_TM, _TN), lambda j, i: (i, j)),
        compiler_params=pltpu.CompilerParams(
            dimension_semantics=("parallel", "parallel"),
            vmem_limit_bytes=64 * 1024 * 1024,
        ),
        cost_estimate=pl.CostEstimate(
            flops=2 * M * _N * _K,
            transcendentals=0,
            bytes_accessed=(
                x2d.size * 4 + _K * _N * 4 + b_sl.size * 4 + M * _N * 4
            ),
        ),
    )(x2d, w_bf, b_sl)

    return out.reshape(*lead_shape, _N)


# speedup vs baseline: 2.8060x; 1.0112x over previous
import jax
import jax.numpy as jnp
from jax.experimental import pallas as pl
from jax.experimental.pallas import tpu as pltpu

_K = 1024
_N = 32000
_TM = 1024
_TN = 3200


def _matmul_bias_kernel(x_ref, w_ref, b_ref, o_ref):
    x_bf = x_ref[...].astype(jnp.bfloat16)
    w_bf = w_ref[...].astype(jnp.bfloat16)
    acc = jnp.dot(x_bf, w_bf, preferred_element_type=jnp.float32)
    o_ref[...] = acc + b_ref[...]


def kernel(x, w_kn, b):
    lead_shape = x.shape[:-1]
    x2d = x.reshape(-1, _K)
    M = x2d.shape[0]
    w_bf = w_kn
    b_sl = b[:, :_N]

    grid = (_N // _TN, M // _TM)

    out = pl.pallas_call(
        _matmul_bias_kernel,
        out_shape=jax.ShapeDtypeStruct((M, _N), jnp.float32),
        grid=grid,
        in_specs=[
            pl.BlockSpec((_TM, _K), lambda j, i: (i, 0)),
            pl.BlockSpec((_K, _TN), lambda j, i: (0, j)),
            pl.BlockSpec((1, _TN), lambda j, i: (0, j)),
        ],
        out_specs=pl.BlockSpec((_TM, _TN), lambda j, i: (i, j)),
        compiler_params=pltpu.CompilerParams(
            dimension_semantics=("parallel", "arbitrary"),
            vmem_limit_bytes=64 * 1024 * 1024,
        ),
        cost_estimate=pl.CostEstimate(
            flops=2 * M * _N * _K,
            transcendentals=0,
            bytes_accessed=(
                x2d.size * 4 + _K * _N * 4 + b_sl.size * 4 + M * _N * 4
            ),
        ),
    )(x2d, w_bf, b_sl)

    return out.reshape(*lead_shape, _N)
